# baseline (speedup 1.0000x reference)
import functools

import jax
import jax.numpy as jnp
from jax import lax
from jax.experimental import pallas as pl
from jax.experimental.pallas import tpu as pltpu
from jax.experimental.pallas import tpu_sc as plsc

VOCAB = 1000
N_EMBD = 32
T = 8
ROWS = 4096 * 8
NW = 32
ROWS_PER_W = ROWS // NW
CHUNK = 128
NCHUNK = ROWS_PER_W // CHUNK
BLK = 2048
GRID = ROWS // BLK
G = BLK // 16
PACK = BLK * VOCAB // 128
PROWS = ROWS * VOCAB // 128


def _sc_gather_kernel(table_hbm, idx_hbm, out_hbm, idx_v, rows_v, sem):
    wid = lax.axis_index("s") * 2 + lax.axis_index("c")
    base = wid * NCHUNK
    pltpu.sync_copy(idx_hbm.at[pl.ds(base, NCHUNK)], idx_v)
    for j in range(NCHUNK):
        pltpu.async_copy(table_hbm.at[idx_v.at[j]], rows_v.at[j], sem).wait()
        pltpu.sync_copy(
            rows_v.at[j],
            out_hbm.at[pl.ds(wid * ROWS_PER_W + j * CHUNK, CHUNK)],
        )


@jax.jit
def _sc_gather(tok_table, idx2):
    mesh = plsc.VectorSubcoreMesh(core_axis_name="c", subcore_axis_name="s")
    return pl.kernel(
        _sc_gather_kernel,
        mesh=mesh,
        out_type=jax.ShapeDtypeStruct((ROWS, N_EMBD), jnp.float32),
        scratch_types=[
            pltpu.VMEM((NCHUNK, CHUNK), jnp.int32),
            pltpu.VMEM((NCHUNK, CHUNK, N_EMBD), jnp.float32),
            pltpu.SemaphoreType.DMA,
        ],
        compiler_params=pltpu.CompilerParams(use_tc_tiling_on_sc=False),
    )(tok_table, idx2)


def _repack(logits):
    l3 = logits.reshape(G, 16, VOCAB)
    canv = []
    for k in range(16):
        s = (VOCAB * k) % 128
        nrows = (s + VOCAB + 127) // 128
        pieces = [l3[:, k, :]]
        if s:
            pieces.insert(0, jnp.zeros((G, s), jnp.float32))
        tail = nrows * 128 - VOCAB - s
        if tail:
            pieces.append(jnp.zeros((G, tail), jnp.float32))
        canv.append(jnp.concatenate(pieces, axis=1))
    out = [canv[0][:, : 7 * 128]]
    for k in range(1, 16):
        out.append(canv[k - 1][:, -128:] + canv[k][:, :128])
        last = canv[k].shape[1] - (128 if k < 15 else 0)
        out.append(canv[k][:, 128:last])
    flat = jnp.concatenate(out, axis=1)
    return flat.reshape(G, 125, 128).reshape(PACK, 128)


def _tc_head_kernel(x_ref, pos_ref, w_ref, b_ref, t_ref, logits_ref, loss_ref):
    i = pl.program_id(0)
    x = x_ref[...]
    xp = x.reshape(BLK // T, T, N_EMBD) + pos_ref[...][None, :, :]
    xp = xp.reshape(BLK, N_EMBD)
    logits = (
        jnp.dot(xp, w_ref[...], preferred_element_type=jnp.float32,
                precision=lax.Precision.DEFAULT)
        + b_ref[...]
    )
    logits_ref[...] = _repack(logits)

    rowmax = jnp.max(logits, axis=1, keepdims=True)
    se = jnp.sum(jnp.exp(logits - rowmax), axis=1)
    viota = lax.broadcasted_iota(jnp.int32, (BLK, VOCAB), 1)
    tmask = viota == t_ref[...]
    tlogit = jnp.sum(jnp.where(tmask, logits, 0.0), axis=1)
    bs = jnp.sum(jnp.log(se) + rowmax[:, 0] - tlogit).reshape(1, 1)

    @pl.when(i == 0)
    def _init():
        loss_ref[...] = jnp.zeros((1, 1), jnp.float32)

    loss_ref[...] += bs

    @pl.when(i == pl.num_programs(0) - 1)
    def _fin():
        loss_ref[...] = loss_ref[...] / ROWS


@jax.jit
def _tc_head(x, pos_table, W, b2, t2):
    return pl.pallas_call(
        _tc_head_kernel,
        grid=(GRID,),
        in_specs=[
            pl.BlockSpec((BLK, N_EMBD), lambda i: (i, 0)),
            pl.BlockSpec((T, N_EMBD), lambda i: (0, 0)),
            pl.BlockSpec((N_EMBD, VOCAB), lambda i: (0, 0)),
            pl.BlockSpec((1, VOCAB), lambda i: (0, 0)),
            pl.BlockSpec((BLK, 1), lambda i: (i, 0)),
        ],
        out_specs=[
            pl.BlockSpec((PACK, 128), lambda i: (i, 0)),
            pl.BlockSpec((1, 1), lambda i: (0, 0)),
        ],
        out_shape=[
            jax.ShapeDtypeStruct((PROWS, 128), jnp.float32),
            jax.ShapeDtypeStruct((1, 1), jnp.float32),
        ],
    )(x, pos_table, W, b2, t2)


def kernel(idx, targets, tok_table, pos_table, W, b):
    idx2 = idx.reshape(NW * NCHUNK, CHUNK).astype(jnp.int32)
    x = _sc_gather(tok_table, idx2)
    t2 = targets.reshape(ROWS, 1).astype(jnp.int32)
    packed, loss = _tc_head(x, pos_table, W, b.reshape(1, VOCAB), t2)
    return (packed.reshape(ROWS, VOCAB), loss[0, 0])


# speedup vs baseline: 2.2393x; 2.2393x over previous
import functools

import jax
import jax.numpy as jnp
from jax import lax
from jax.experimental import pallas as pl
from jax.experimental.pallas import tpu as pltpu
from jax.experimental.pallas import tpu_sc as plsc

VOCAB = 1000
VPAD = 1024
N_EMBD = 32
T = 8
ROWS = 4096 * 8
NW = 32
ROWS_PER_W = ROWS // NW
CHUNK = 128
NCHUNK = ROWS_PER_W // CHUNK
BLK = 4096
GRID = ROWS // BLK


def _sc_gather_kernel(table_hbm, idx_hbm, out_hbm, idx_v, rows_v, sem):
    wid = lax.axis_index("s") * 2 + lax.axis_index("c")
    base = wid * NCHUNK
    pltpu.sync_copy(idx_hbm.at[pl.ds(base, NCHUNK)], idx_v)
    for j in range(NCHUNK):
        pltpu.async_copy(table_hbm.at[idx_v.at[j]], rows_v.at[j], sem).wait()
        pltpu.sync_copy(
            rows_v.at[j],
            out_hbm.at[pl.ds(wid * ROWS_PER_W + j * CHUNK, CHUNK)],
        )


@jax.jit
def _sc_gather(tok_table, idx2):
    mesh = plsc.VectorSubcoreMesh(core_axis_name="c", subcore_axis_name="s")
    return pl.kernel(
        _sc_gather_kernel,
        mesh=mesh,
        out_type=jax.ShapeDtypeStruct((ROWS, N_EMBD), jnp.float32),
        scratch_types=[
            pltpu.VMEM((NCHUNK, CHUNK), jnp.int32),
            pltpu.VMEM((NCHUNK, CHUNK, N_EMBD), jnp.float32),
            pltpu.SemaphoreType.DMA,
        ],
        compiler_params=pltpu.CompilerParams(use_tc_tiling_on_sc=False),
    )(tok_table, idx2)


def _tc_head_kernel(x_ref, pos_ref, w_ref, b_ref, t_ref, logits_ref, loss_ref):
    i = pl.program_id(0)
    x = x_ref[...]
    xp = x.reshape(BLK // T, T, N_EMBD) + pos_ref[...][None, :, :]
    xp = xp.reshape(BLK, N_EMBD)
    logits = (
        jnp.dot(xp, w_ref[...], preferred_element_type=jnp.float32,
                precision=lax.Precision.DEFAULT)
        + b_ref[...]
    )
    logits_ref[...] = logits

    viota = lax.broadcasted_iota(jnp.int32, (BLK, VPAD), 1)
    valid = viota < VOCAB
    neg = jnp.where(valid, logits, -jnp.inf)
    rowmax = jnp.max(neg, axis=1, keepdims=True)
    se = jnp.sum(jnp.where(valid, jnp.exp(logits - rowmax), 0.0), axis=1)
    tmask = viota == t_ref[...]
    tlogit = jnp.sum(jnp.where(tmask, logits, 0.0), axis=1)
    bs = jnp.sum(jnp.log(se) + rowmax[:, 0] - tlogit).reshape(1, 1)

    @pl.when(i == 0)
    def _init():
        loss_ref[...] = jnp.zeros((1, 1), jnp.float32)

    loss_ref[...] += bs

    @pl.when(i == pl.num_programs(0) - 1)
    def _fin():
        loss_ref[...] = loss_ref[...] / ROWS


@jax.jit
def _tc_head(x, pos_table, Wp, bp, t2):
    return pl.pallas_call(
        _tc_head_kernel,
        grid=(GRID,),
        in_specs=[
            pl.BlockSpec((BLK, N_EMBD), lambda i: (i, 0)),
            pl.BlockSpec((T, N_EMBD), lambda i: (0, 0)),
            pl.BlockSpec((N_EMBD, VPAD), lambda i: (0, 0)),
            pl.BlockSpec((1, VPAD), lambda i: (0, 0)),
            pl.BlockSpec((BLK, 1), lambda i: (i, 0)),
        ],
        out_specs=[
            pl.BlockSpec((BLK, VPAD), lambda i: (i, 0)),
            pl.BlockSpec((1, 1), lambda i: (0, 0)),
        ],
        out_shape=[
            jax.ShapeDtypeStruct((ROWS, VPAD), jnp.float32),
            jax.ShapeDtypeStruct((1, 1), jnp.float32),
        ],
    )(x, pos_table, Wp, bp, t2)


def kernel(idx, targets, tok_table, pos_table, W, b):
    idx2 = idx.reshape(NW * NCHUNK, CHUNK).astype(jnp.int32)
    x = _sc_gather(tok_table, idx2)
    t2 = targets.reshape(ROWS, 1).astype(jnp.int32)
    Wp = jnp.pad(W, ((0, 0), (0, VPAD - VOCAB)))
    bp = jnp.pad(b, (0, VPAD - VOCAB)).reshape(1, VPAD)
    padded, loss = _tc_head(x, pos_table, Wp, bp, t2)
    return (padded[:, :VOCAB], loss[0, 0])
